# baseline (speedup 1.0000x reference)
import functools

import jax
import jax.numpy as jnp
from jax import lax
from jax.experimental import pallas as pl
from jax.experimental.pallas import tpu as pltpu
from jax.experimental.pallas import tpu_sc as plsc


_NC = 2
_NS = 16
_NW = _NC * _NS
_BB = 128


def _emb_kernel(n_b, seq_len, d, seq_hbm, table_hbm, pos_hbm, out_hbm,
                idx_v, rows_v, pos_v, trans_v, gsems, wsems):
    cb = lax.axis_index("s") * _NC + lax.axis_index("c")
    n_lt = seq_len // 8
    rd8 = d // 8

    pltpu.sync_copy(seq_hbm.at[:, cb], idx_v)
    pltpu.sync_copy(pos_hbm, pos_v)

    bidx = [lax.iota(jnp.int32, 16) + 16 * k for k in range(8)]

    for lr in range(8):
        pltpu.async_copy(table_hbm.at[idx_v.at[0, lr]], rows_v.at[lr],
                         gsems[lr])

    @pl.loop(0, n_lt)
    def _lts(lt):
        for lr in range(8):
            l = lt * 8 + lr
            tb = lr % 2

            pltpu.make_async_copy(table_hbm.at[idx_v.at[lt, lr]],
                                  rows_v.at[lr], gsems[lr]).wait()

            @pl.when(l >= 2)
            def _():
                pltpu.make_async_copy(trans_v.at[tb],
                                      out_hbm.at[0, :, cb], wsems[tb]).wait()

            lsplat = jnp.full((16,), l, jnp.int32)

            @pl.loop(0, rd8)
            def _rds(rd):
                for d8 in range(8):
                    dd = rd * 8 + d8
                    dsplat = jnp.full((16,), dd, jnp.int32)
                    p = plsc.load_gather(pos_v, [lsplat, dsplat])
                    for k in range(8):
                        g = plsc.load_gather(rows_v.at[lr], [bidx[k], dsplat])
                        trans_v[tb, rd, d8, pl.ds(k * 16, 16)] = g + p

            pltpu.async_copy(trans_v.at[tb], out_hbm.at[l, :, cb], wsems[tb])

            @pl.when(lt + 1 < n_lt)
            def _():
                pltpu.async_copy(table_hbm.at[idx_v.at[lt + 1, lr]],
                                 rows_v.at[lr], gsems[lr])

    for tb in range(2):
        pltpu.make_async_copy(trans_v.at[tb], out_hbm.at[0, :, cb],
                              wsems[tb]).wait()


def kernel(item_seq, item_table, token_mask, pos_table):
    del token_mask
    n_b, seq_len = item_seq.shape
    d = item_table.shape[1]

    st = jnp.transpose(item_seq.astype(jnp.int32))
    sr = jnp.reshape(st, (seq_len // 8, 8, n_b // _BB, _BB))
    seq_native = jnp.transpose(sr, (0, 2, 1, 3))

    mesh = plsc.VectorSubcoreMesh(core_axis_name="c", subcore_axis_name="s")
    fn = pl.kernel(
        functools.partial(_emb_kernel, n_b, seq_len, d),
        out_type=jax.ShapeDtypeStruct(
            (seq_len, d // 8, n_b // _BB, 8, _BB), jnp.float32),
        mesh=mesh,
        scratch_types=[
            pltpu.VMEM((seq_len // 8, 8, _BB), jnp.int32),
            pltpu.VMEM((8, _BB, d), jnp.float32),
            pltpu.VMEM((seq_len, d), jnp.float32),
            pltpu.VMEM((2, d // 8, 8, _BB), jnp.float32),
            [pltpu.SemaphoreType.DMA] * 8,
            [pltpu.SemaphoreType.DMA] * 2,
        ],
        compiler_params=pltpu.CompilerParams(use_tc_tiling_on_sc=False,
                                            needs_layout_passes=False),
    )
    out_lin = fn(seq_native, item_table, pos_table)

    t = jnp.transpose(out_lin, (2, 4, 0, 1, 3))
    return jnp.reshape(t, (n_b, seq_len, d))


# speedup vs baseline: 1.7762x; 1.7762x over previous
import functools

import jax
import jax.numpy as jnp
from jax import lax
from jax.experimental import pallas as pl
from jax.experimental.pallas import tpu as pltpu
from jax.experimental.pallas import tpu_sc as plsc


_NC = 2
_NS = 16
_NW = _NC * _NS
_BB = 128


def _emb_kernel(n_b, seq_len, d, seq_hbm, table_hbm, pos_hbm, out_hbm,
                idx_v, rows_v, pos_v, trans_v, gsems, wsems):
    cb = lax.axis_index("s") * _NC + lax.axis_index("c")
    n_lt = seq_len // 8

    pltpu.sync_copy(seq_hbm.at[:, cb], idx_v)
    pltpu.sync_copy(pos_hbm, pos_v)

    nvec = d // 16
    dbase = [lax.iota(jnp.int32, 16) + 16 * j for j in range(nvec)]

    for lr in range(8):
        pltpu.async_copy(table_hbm.at[idx_v.at[0, lr]], rows_v.at[lr],
                         gsems[lr])

    @pl.loop(0, n_lt)
    def _lts(lt):
        for lr in range(8):
            l = lt * 8 + lr
            tb = lr % 2

            pltpu.make_async_copy(table_hbm.at[idx_v.at[lt, lr]],
                                  rows_v.at[lr], gsems[lr]).wait()

            @pl.when(l >= 2)
            def _():
                pltpu.make_async_copy(trans_v.at[tb, :, :, pl.ds(0, _BB)],
                                      out_hbm.at[0, :, cb], wsems[tb]).wait()

            prow = [pos_v[l, pl.ds(16 * j, 16)] for j in range(nvec)]

            @pl.loop(0, _BB, unroll=4)
            def _bs(b):
                bsplat = jnp.full((16,), b, jnp.int32)
                for j in range(nvec):
                    dvec = dbase[j]
                    v = rows_v[lr, b, pl.ds(16 * j, 16)] + prow[j]
                    plsc.store_scatter(trans_v.at[tb],
                                       [dvec >> 3, dvec & 7, bsplat], v)

            pltpu.async_copy(trans_v.at[tb, :, :, pl.ds(0, _BB)],
                             out_hbm.at[l, :, cb], wsems[tb])

            @pl.when(lt + 1 < n_lt)
            def _():
                pltpu.async_copy(table_hbm.at[idx_v.at[lt + 1, lr]],
                                 rows_v.at[lr], gsems[lr])

    for tb in range(2):
        pltpu.make_async_copy(trans_v.at[tb, :, :, pl.ds(0, _BB)],
                              out_hbm.at[0, :, cb], wsems[tb]).wait()


def kernel(item_seq, item_table, token_mask, pos_table):
    del token_mask
    n_b, seq_len = item_seq.shape
    d = item_table.shape[1]

    st = jnp.transpose(item_seq.astype(jnp.int32))
    sr = jnp.reshape(st, (seq_len // 8, 8, n_b // _BB, _BB))
    seq_native = jnp.transpose(sr, (0, 2, 1, 3))

    mesh = plsc.VectorSubcoreMesh(core_axis_name="c", subcore_axis_name="s")
    fn = pl.kernel(
        functools.partial(_emb_kernel, n_b, seq_len, d),
        out_type=jax.ShapeDtypeStruct(
            (seq_len, d // 8, n_b // _BB, 8, _BB), jnp.float32),
        mesh=mesh,
        scratch_types=[
            pltpu.VMEM((seq_len // 8, 8, _BB), jnp.int32),
            pltpu.VMEM((8, _BB, d), jnp.float32),
            pltpu.VMEM((seq_len, d), jnp.float32),
            pltpu.VMEM((2, d // 8, 8, _BB + 5), jnp.float32),
            [pltpu.SemaphoreType.DMA] * 8,
            [pltpu.SemaphoreType.DMA] * 2,
        ],
        compiler_params=pltpu.CompilerParams(use_tc_tiling_on_sc=False,
                                            needs_layout_passes=False),
    )
    out_lin = fn(seq_native, item_table, pos_table)

    t = jnp.transpose(out_lin, (2, 4, 0, 1, 3))
    return jnp.reshape(t, (n_b, seq_len, d))
